# Initial kernel scaffold; baseline (speedup 1.0000x reference)
import functools

import jax
import jax.numpy as jnp
from jax import lax
from jax.experimental import pallas as pl
from jax.experimental.pallas import tpu as pltpu
from jax.experimental.pallas import tpu_sc as plsc

N = 10000
E = 320000
F = 128
OUT_F = 64
NC, NS = 2, 16
NW = NC * NS
CH = 80
NCHUNK = E // (NW * CH)
RPT = N // NS
ZR = 25
DEG_W = 16

_mesh = plsc.VectorSubcoreMesh(
    core_axis_name="c", subcore_axis_name="s", num_cores=NC, num_subcores=NS)


@functools.partial(
    pl.kernel,
    out_type=jax.ShapeDtypeStruct((NC, N, DEG_W), jnp.float32),
    mesh=_mesh,
    scratch_types=[
        pltpu.VMEM((NCHUNK, CH), jnp.int32),
        pltpu.VMEM((CH, DEG_W), jnp.float32),
        pltpu.VMEM((ZR, DEG_W), jnp.float32),
        pltpu.VMEM_SHARED((N, DEG_W), jnp.float32),
        pltpu.SemaphoreType.DMA,
    ],
)
def _deg_kernel(dst_hbm, out_hbm, dst_v, ones_v, zero_v, acc, sem):
    c = lax.axis_index("c")
    s = lax.axis_index("s")
    wid = c * NS + s
    for r in range(CH):
        ones_v[r, :] = jnp.ones((DEG_W,), jnp.float32)
    for r in range(ZR):
        zero_v[r, :] = jnp.zeros((DEG_W,), jnp.float32)
    for k in range(RPT // ZR):
        pltpu.sync_copy(zero_v, acc.at[pl.ds(s * RPT + k * ZR, ZR)])
    pltpu.sync_copy(dst_hbm.at[pl.ds(wid * NCHUNK, NCHUNK)], dst_v)
    plsc.subcore_barrier()

    def body(j, carry):
        pltpu.sync_copy(ones_v, acc.at[dst_v.at[j]], add=True)
        return carry

    lax.fori_loop(0, NCHUNK, body, 0)
    plsc.subcore_barrier()
    pltpu.sync_copy(acc.at[pl.ds(s * RPT, RPT)],
                    out_hbm.at[c, pl.ds(s * RPT, RPT)])


@functools.partial(
    pl.kernel,
    out_type=jax.ShapeDtypeStruct((NC, N, F), jnp.float32),
    mesh=_mesh,
    scratch_types=[
        pltpu.VMEM((NCHUNK, CH), jnp.int32),
        pltpu.VMEM((NCHUNK, CH), jnp.int32),
        pltpu.VMEM((CH, F), jnp.float32),
        pltpu.VMEM((ZR, F), jnp.float32),
        pltpu.VMEM_SHARED((N, F), jnp.float32),
        pltpu.SemaphoreType.DMA,
    ],
)
def _agg_kernel(q_hbm, src_hbm, dst_hbm, out_hbm,
                src_v, dst_v, rows_v, zero_v, acc, sem):
    c = lax.axis_index("c")
    s = lax.axis_index("s")
    wid = c * NS + s
    for r in range(ZR):
        for j in range(F // 16):
            zero_v[r, pl.ds(j * 16, 16)] = jnp.zeros((16,), jnp.float32)
    for k in range(RPT // ZR):
        pltpu.sync_copy(zero_v, acc.at[pl.ds(s * RPT + k * ZR, ZR)])
    pltpu.sync_copy(src_hbm.at[pl.ds(wid * NCHUNK, NCHUNK)], src_v)
    pltpu.sync_copy(dst_hbm.at[pl.ds(wid * NCHUNK, NCHUNK)], dst_v)
    plsc.subcore_barrier()

    def body(j, carry):
        pltpu.async_copy(q_hbm.at[src_v.at[j]], rows_v, sem).wait()
        pltpu.sync_copy(rows_v, acc.at[dst_v.at[j]], add=True)
        return carry

    lax.fori_loop(0, NCHUNK, body, 0)
    plsc.subcore_barrier()
    pltpu.sync_copy(acc.at[pl.ds(s * RPT, RPT)],
                    out_hbm.at[c, pl.ds(s * RPT, RPT)])


BR = 400
GRID = N // BR


def _mm_body(x_ref, w_ref, o_ref):
    o_ref[...] = jnp.dot(x_ref[...], w_ref[...],
                         preferred_element_type=jnp.float32)


def _matmul(x, w):
    return pl.pallas_call(
        _mm_body,
        grid=(GRID,),
        in_specs=[
            pl.BlockSpec((BR, F), lambda i: (i, 0)),
            pl.BlockSpec((F, F), lambda i: (0, 0)),
        ],
        out_specs=pl.BlockSpec((BR, F), lambda i: (i, 0)),
        out_shape=jax.ShapeDtypeStruct((N, F), jnp.float32),
    )(x, w)


def _scale_body(p_ref, d0_ref, d1_ref, q_ref, dv_ref):
    deg = 1.0 + d0_ref[...] + d1_ref[...]
    dinv = lax.rsqrt(deg)
    dv_ref[...] = dinv
    q_ref[...] = p_ref[...] * dinv[:, 0:1]


def _scale(p, d0, d1):
    return pl.pallas_call(
        _scale_body,
        grid=(GRID,),
        in_specs=[
            pl.BlockSpec((BR, F), lambda i: (i, 0)),
            pl.BlockSpec((BR, DEG_W), lambda i: (i, 0)),
            pl.BlockSpec((BR, DEG_W), lambda i: (i, 0)),
        ],
        out_specs=[
            pl.BlockSpec((BR, F), lambda i: (i, 0)),
            pl.BlockSpec((BR, DEG_W), lambda i: (i, 0)),
        ],
        out_shape=[
            jax.ShapeDtypeStruct((N, F), jnp.float32),
            jax.ShapeDtypeStruct((N, DEG_W), jnp.float32),
        ],
    )(p, d0, d1)


def _mid_body(s0_ref, s1_ref, q_ref, dv_ref, b_ref, w_ref, o_ref):
    dv = dv_ref[:, 0:1]
    h = dv * (s0_ref[...] + s1_ref[...] + q_ref[...]) + b_ref[...]
    h = jnp.maximum(h, 0.0)
    o_ref[...] = jnp.dot(h, w_ref[...],
                         preferred_element_type=jnp.float32) * dv


def _mid(s0, s1, q, dv, b, w):
    return pl.pallas_call(
        _mid_body,
        grid=(GRID,),
        in_specs=[
            pl.BlockSpec((BR, F), lambda i: (i, 0)),
            pl.BlockSpec((BR, F), lambda i: (i, 0)),
            pl.BlockSpec((BR, F), lambda i: (i, 0)),
            pl.BlockSpec((BR, DEG_W), lambda i: (i, 0)),
            pl.BlockSpec((1, F), lambda i: (0, 0)),
            pl.BlockSpec((F, F), lambda i: (0, 0)),
        ],
        out_specs=pl.BlockSpec((BR, F), lambda i: (i, 0)),
        out_shape=jax.ShapeDtypeStruct((N, F), jnp.float32),
    )(s0, s1, q, dv, b, w)


def _head_body(s0_ref, s1_ref, q_ref, dv_ref, b_ref, wc_ref, bc_ref,
               logits_ref, emb_ref, soft_ref, hard_ref):
    dv = dv_ref[:, 0:1]
    emb = dv * (s0_ref[...] + s1_ref[...] + q_ref[...]) + b_ref[...]
    emb_ref[...] = emb
    logits = jnp.dot(emb, wc_ref[...],
                     preferred_element_type=jnp.float32) + bc_ref[...]
    logits_ref[...] = logits
    m = jnp.max(logits, axis=1, keepdims=True)
    ex = jnp.exp(logits - m)
    soft = ex / jnp.sum(ex, axis=1, keepdims=True)
    soft_ref[...] = soft
    smax = jnp.max(soft, axis=1, keepdims=True)
    ids = lax.broadcasted_iota(jnp.int32, (BR, OUT_F), 1)
    cand = jnp.where(soft >= smax, ids, OUT_F)
    hard_ref[...] = jnp.min(cand, axis=1, keepdims=True)


def _head(s0, s1, q, dv, b, wc, bc):
    return pl.pallas_call(
        _head_body,
        grid=(GRID,),
        in_specs=[
            pl.BlockSpec((BR, F), lambda i: (i, 0)),
            pl.BlockSpec((BR, F), lambda i: (i, 0)),
            pl.BlockSpec((BR, F), lambda i: (i, 0)),
            pl.BlockSpec((BR, DEG_W), lambda i: (i, 0)),
            pl.BlockSpec((1, F), lambda i: (0, 0)),
            pl.BlockSpec((F, OUT_F), lambda i: (0, 0)),
            pl.BlockSpec((1, OUT_F), lambda i: (0, 0)),
        ],
        out_specs=[
            pl.BlockSpec((BR, OUT_F), lambda i: (i, 0)),
            pl.BlockSpec((BR, F), lambda i: (i, 0)),
            pl.BlockSpec((BR, OUT_F), lambda i: (i, 0)),
            pl.BlockSpec((BR, 1), lambda i: (i, 0)),
        ],
        out_shape=[
            jax.ShapeDtypeStruct((N, OUT_F), jnp.float32),
            jax.ShapeDtypeStruct((N, F), jnp.float32),
            jax.ShapeDtypeStruct((N, OUT_F), jnp.float32),
            jax.ShapeDtypeStruct((N, 1), jnp.int32),
        ],
    )(s0, s1, q, dv, b, wc, bc)


def kernel(x, edge_index, W1, b1, W2, b2, Wc, bc):
    src2d = edge_index[0].astype(jnp.int32).reshape(E // CH, CH)
    dst2d = edge_index[1].astype(jnp.int32).reshape(E // CH, CH)
    b1r = b1.reshape(1, F)
    b2r = b2.reshape(1, F)
    bcr = bc.reshape(1, OUT_F)

    degp = _deg_kernel(dst2d)
    p1 = _matmul(x, W1)
    q1, dv = _scale(p1, degp[0], degp[1])
    s1 = _agg_kernel(q1, src2d, dst2d)
    q2 = _mid(s1[0], s1[1], q1, dv, b1r, W2)
    s2 = _agg_kernel(q2, src2d, dst2d)
    logits, emb, soft, hard = _head(s2[0], s2[1], q2, dv, b2r, Wc, bcr)
    return (logits, emb, soft, hard.reshape(N))


# speedup vs baseline: 15.3944x; 15.3944x over previous
import functools

import jax
import jax.numpy as jnp
from jax import lax
from jax.experimental import pallas as pl
from jax.experimental.pallas import tpu as pltpu
from jax.experimental.pallas import tpu_sc as plsc

N = 10000
E = 320000
F = 128
OUT_F = 64
NC, NS = 2, 16
NW = NC * NS
CH = 80
NCHUNK = E // (NW * CH)
RPT = N // NS
ZR = 25
DEG_W = 16

_mesh = plsc.VectorSubcoreMesh(
    core_axis_name="c", subcore_axis_name="s", num_cores=NC, num_subcores=NS)


@functools.partial(
    pl.kernel,
    out_type=jax.ShapeDtypeStruct((NC, NS, RPT, F), jnp.float32),
    mesh=_mesh,
    scratch_types=[
        pltpu.VMEM((NCHUNK, CH), jnp.int32),
        pltpu.VMEM((CH, F), jnp.float32),
        pltpu.VMEM((ZR, F), jnp.float32),
        pltpu.VMEM_SHARED((N, F), jnp.float32),
        pltpu.SemaphoreType.DMA,
    ],
)
def _deg_kernel(dst_hbm, out_hbm, dst_v, ones_v, zero_v, acc, sem):
    c = lax.axis_index("c")
    s = lax.axis_index("s")
    wid = c * NS + s
    for r in range(CH):
        for j in range(F // 16):
            ones_v[r, pl.ds(j * 16, 16)] = jnp.ones((16,), jnp.float32)
    for r in range(ZR):
        for j in range(F // 16):
            zero_v[r, pl.ds(j * 16, 16)] = jnp.zeros((16,), jnp.float32)
    for k in range(RPT // ZR):
        pltpu.sync_copy(zero_v, acc.at[pl.ds(s * RPT + k * ZR, ZR)])
    pltpu.sync_copy(dst_hbm.at[wid], dst_v)
    plsc.subcore_barrier()

    def body(j, carry):
        pltpu.sync_copy(ones_v, acc.at[dst_v.at[j]], add=True)
        return carry

    lax.fori_loop(0, NCHUNK, body, 0)
    plsc.subcore_barrier()
    pltpu.sync_copy(acc.at[pl.ds(s * RPT, RPT)], out_hbm.at[c, s])


@functools.partial(
    pl.kernel,
    out_type=jax.ShapeDtypeStruct((NC, NS, RPT, F), jnp.float32),
    mesh=_mesh,
    scratch_types=[
        pltpu.VMEM((NCHUNK, CH), jnp.int32),
        pltpu.VMEM((NCHUNK, CH), jnp.int32),
        pltpu.VMEM((CH, F), jnp.float32),
        pltpu.VMEM((ZR, F), jnp.float32),
        pltpu.VMEM_SHARED((N, F), jnp.float32),
        pltpu.SemaphoreType.DMA,
    ],
)
def _agg_kernel(q_hbm, src_hbm, dst_hbm, out_hbm,
                src_v, dst_v, rows_v, zero_v, acc, sem):
    c = lax.axis_index("c")
    s = lax.axis_index("s")
    wid = c * NS + s
    for r in range(ZR):
        for j in range(F // 16):
            zero_v[r, pl.ds(j * 16, 16)] = jnp.zeros((16,), jnp.float32)
    for k in range(RPT // ZR):
        pltpu.sync_copy(zero_v, acc.at[pl.ds(s * RPT + k * ZR, ZR)])
    pltpu.sync_copy(src_hbm.at[wid], src_v)
    pltpu.sync_copy(dst_hbm.at[wid], dst_v)
    plsc.subcore_barrier()

    def body(j, carry):
        pltpu.async_copy(q_hbm.at[src_v.at[j]], rows_v, sem).wait()
        pltpu.sync_copy(rows_v, acc.at[dst_v.at[j]], add=True)
        return carry

    lax.fori_loop(0, NCHUNK, body, 0)
    plsc.subcore_barrier()
    pltpu.sync_copy(acc.at[pl.ds(s * RPT, RPT)], out_hbm.at[c, s])


BR = 400
GRID = N // BR


def _mm_body(x_ref, w_ref, o_ref):
    o_ref[...] = jnp.dot(x_ref[...], w_ref[...],
                         preferred_element_type=jnp.float32)


def _matmul(x, w):
    return pl.pallas_call(
        _mm_body,
        grid=(GRID,),
        in_specs=[
            pl.BlockSpec((BR, F), lambda i: (i, 0)),
            pl.BlockSpec((F, F), lambda i: (0, 0)),
        ],
        out_specs=pl.BlockSpec((BR, F), lambda i: (i, 0)),
        out_shape=jax.ShapeDtypeStruct((N, F), jnp.float32),
    )(x, w)


def _scale_body(p_ref, d0_ref, d1_ref, q_ref, dv_ref):
    deg = 1.0 + d0_ref[...] + d1_ref[...]
    dinv = lax.rsqrt(deg)
    dv_ref[...] = dinv[:, 0:DEG_W]
    q_ref[...] = p_ref[...] * dinv


def _scale(p, d0, d1):
    return pl.pallas_call(
        _scale_body,
        grid=(GRID,),
        in_specs=[
            pl.BlockSpec((BR, F), lambda i: (i, 0)),
            pl.BlockSpec((BR, F), lambda i: (i, 0)),
            pl.BlockSpec((BR, F), lambda i: (i, 0)),
        ],
        out_specs=[
            pl.BlockSpec((BR, F), lambda i: (i, 0)),
            pl.BlockSpec((BR, DEG_W), lambda i: (i, 0)),
        ],
        out_shape=[
            jax.ShapeDtypeStruct((N, F), jnp.float32),
            jax.ShapeDtypeStruct((N, DEG_W), jnp.float32),
        ],
    )(p, d0, d1)


def _mid_body(s0_ref, s1_ref, q_ref, dv_ref, b_ref, w_ref, o_ref):
    dv = dv_ref[:, 0:1]
    h = dv * (s0_ref[...] + s1_ref[...] + q_ref[...]) + b_ref[...]
    h = jnp.maximum(h, 0.0)
    o_ref[...] = jnp.dot(h, w_ref[...],
                         preferred_element_type=jnp.float32) * dv


def _mid(s0, s1, q, dv, b, w):
    return pl.pallas_call(
        _mid_body,
        grid=(GRID,),
        in_specs=[
            pl.BlockSpec((BR, F), lambda i: (i, 0)),
            pl.BlockSpec((BR, F), lambda i: (i, 0)),
            pl.BlockSpec((BR, F), lambda i: (i, 0)),
            pl.BlockSpec((BR, DEG_W), lambda i: (i, 0)),
            pl.BlockSpec((1, F), lambda i: (0, 0)),
            pl.BlockSpec((F, F), lambda i: (0, 0)),
        ],
        out_specs=pl.BlockSpec((BR, F), lambda i: (i, 0)),
        out_shape=jax.ShapeDtypeStruct((N, F), jnp.float32),
    )(s0, s1, q, dv, b, w)


def _head_body(s0_ref, s1_ref, q_ref, dv_ref, b_ref, wc_ref, bc_ref,
               logits_ref, emb_ref, soft_ref, hard_ref):
    dv = dv_ref[:, 0:1]
    emb = dv * (s0_ref[...] + s1_ref[...] + q_ref[...]) + b_ref[...]
    emb_ref[...] = emb
    logits = jnp.dot(emb, wc_ref[...],
                     preferred_element_type=jnp.float32) + bc_ref[...]
    logits_ref[...] = logits
    m = jnp.max(logits, axis=1, keepdims=True)
    ex = jnp.exp(logits - m)
    soft = ex / jnp.sum(ex, axis=1, keepdims=True)
    soft_ref[...] = soft
    smax = jnp.max(soft, axis=1, keepdims=True)
    ids = lax.broadcasted_iota(jnp.int32, (BR, OUT_F), 1)
    cand = jnp.where(soft >= smax, ids, OUT_F)
    hard_ref[...] = jnp.min(cand, axis=1, keepdims=True)


def _head(s0, s1, q, dv, b, wc, bc):
    return pl.pallas_call(
        _head_body,
        grid=(GRID,),
        in_specs=[
            pl.BlockSpec((BR, F), lambda i: (i, 0)),
            pl.BlockSpec((BR, F), lambda i: (i, 0)),
            pl.BlockSpec((BR, F), lambda i: (i, 0)),
            pl.BlockSpec((BR, DEG_W), lambda i: (i, 0)),
            pl.BlockSpec((1, F), lambda i: (0, 0)),
            pl.BlockSpec((F, OUT_F), lambda i: (0, 0)),
            pl.BlockSpec((1, OUT_F), lambda i: (0, 0)),
        ],
        out_specs=[
            pl.BlockSpec((BR, OUT_F), lambda i: (i, 0)),
            pl.BlockSpec((BR, F), lambda i: (i, 0)),
            pl.BlockSpec((BR, OUT_F), lambda i: (i, 0)),
            pl.BlockSpec((BR, 1), lambda i: (i, 0)),
        ],
        out_shape=[
            jax.ShapeDtypeStruct((N, OUT_F), jnp.float32),
            jax.ShapeDtypeStruct((N, F), jnp.float32),
            jax.ShapeDtypeStruct((N, OUT_F), jnp.float32),
            jax.ShapeDtypeStruct((N, 1), jnp.int32),
        ],
    )(s0, s1, q, dv, b, wc, bc)


def kernel(x, edge_index, W1, b1, W2, b2, Wc, bc):
    src3d = edge_index[0].astype(jnp.int32).reshape(NW, NCHUNK, CH)
    dst3d = edge_index[1].astype(jnp.int32).reshape(NW, NCHUNK, CH)
    b1r = b1.reshape(1, F)
    b2r = b2.reshape(1, F)
    bcr = bc.reshape(1, OUT_F)

    degp = _deg_kernel(dst3d).reshape(NC, N, F)
    p1 = _matmul(x, W1)
    q1, dv = _scale(p1, degp[0], degp[1])
    s1 = _agg_kernel(q1, src3d, dst3d).reshape(NC, N, F)
    q2 = _mid(s1[0], s1[1], q1, dv, b1r, W2)
    s2 = _agg_kernel(q2, src3d, dst3d).reshape(NC, N, F)
    logits, emb, soft, hard = _head(s2[0], s2[1], q2, dv, b2r, Wc, bcr)
    return (logits, emb, soft, hard.reshape(N))


# speedup vs baseline: 20.3228x; 1.3201x over previous
import functools

import jax
import jax.numpy as jnp
from jax import lax
from jax.experimental import pallas as pl
from jax.experimental.pallas import tpu as pltpu
from jax.experimental.pallas import tpu_sc as plsc

N = 10000
E = 320000
F = 128
OUT_F = 64
NC, NS = 2, 16
NW = NC * NS
CH = 80
NCHUNK = E // (NW * CH)
NB = 25
NBLK = NCHUNK // NB
RPT = N // NS
ZR = 25
DEG_W = 16

_mesh = plsc.VectorSubcoreMesh(
    core_axis_name="c", subcore_axis_name="s", num_cores=NC, num_subcores=NS)


@functools.partial(
    pl.kernel,
    out_type=jax.ShapeDtypeStruct((NC, NS, RPT, F), jnp.float32),
    mesh=_mesh,
    scratch_types=[
        pltpu.VMEM((NCHUNK, CH), jnp.int32),
        pltpu.VMEM((CH, F), jnp.float32),
        pltpu.VMEM((ZR, F), jnp.float32),
        pltpu.VMEM_SHARED((N, F), jnp.float32),
        pltpu.SemaphoreType.DMA,
    ],
)
def _deg_kernel(dst_hbm, out_hbm, dst_v, ones_v, zero_v, acc, sem):
    c = lax.axis_index("c")
    s = lax.axis_index("s")
    wid = c * NS + s
    for r in range(CH):
        for j in range(F // 16):
            ones_v[r, pl.ds(j * 16, 16)] = jnp.ones((16,), jnp.float32)
    for r in range(ZR):
        for j in range(F // 16):
            zero_v[r, pl.ds(j * 16, 16)] = jnp.zeros((16,), jnp.float32)
    for k in range(RPT // ZR):
        pltpu.sync_copy(zero_v, acc.at[pl.ds(s * RPT + k * ZR, ZR)])
    pltpu.sync_copy(dst_hbm.at[wid], dst_v)
    plsc.subcore_barrier()

    def body(j, carry):
        pltpu.sync_copy(ones_v, acc.at[dst_v.at[j]], add=True)
        return carry

    lax.fori_loop(0, NCHUNK, body, 0)
    plsc.subcore_barrier()
    pltpu.sync_copy(acc.at[pl.ds(s * RPT, RPT)], out_hbm.at[c, s])


@functools.partial(
    pl.kernel,
    out_type=jax.ShapeDtypeStruct((NC, NS, RPT, F), jnp.float32),
    mesh=_mesh,
    scratch_types=[
        pltpu.VMEM((NB, CH), jnp.int32),
        pltpu.VMEM((NB, CH), jnp.int32),
        pltpu.VMEM((CH, F), jnp.float32),
        pltpu.VMEM((CH, F), jnp.float32),
        pltpu.VMEM((ZR, F), jnp.float32),
        pltpu.VMEM_SHARED((N, F), jnp.float32),
        pltpu.SemaphoreType.DMA,
        pltpu.SemaphoreType.DMA,
    ],
)
def _agg_kernel(q_hbm, src_hbm, dst_hbm, out_hbm,
                src_v, dst_v, rows0, rows1, zero_v, acc, sem0, sem1):
    c = lax.axis_index("c")
    s = lax.axis_index("s")
    wid = c * NS + s
    for r in range(ZR):
        for j in range(F // 16):
            zero_v[r, pl.ds(j * 16, 16)] = jnp.zeros((16,), jnp.float32)
    for k in range(RPT // ZR):
        pltpu.sync_copy(zero_v, acc.at[pl.ds(s * RPT + k * ZR, ZR)])
    plsc.subcore_barrier()

    def blk(kb, carry):
        pltpu.sync_copy(src_hbm.at[wid, kb], src_v)
        pltpu.sync_copy(dst_hbm.at[wid, kb], dst_v)
        pltpu.async_copy(q_hbm.at[src_v.at[0]], rows0, sem0)

        def body(m, carry2):
            a = 2 * m
            b = a + 1
            pltpu.async_copy(q_hbm.at[src_v.at[b]], rows1, sem1)
            pltpu.make_async_copy(q_hbm.at[src_v.at[a]], rows0, sem0).wait()
            pltpu.sync_copy(rows0, acc.at[dst_v.at[a]], add=True)
            pltpu.async_copy(q_hbm.at[src_v.at[a + 2]], rows0, sem0)
            pltpu.make_async_copy(q_hbm.at[src_v.at[b]], rows1, sem1).wait()
            pltpu.sync_copy(rows1, acc.at[dst_v.at[b]], add=True)
            return carry2

        lax.fori_loop(0, (NB - 1) // 2, body, 0)
        last = NB - 1
        pltpu.make_async_copy(q_hbm.at[src_v.at[last]], rows0, sem0).wait()
        pltpu.sync_copy(rows0, acc.at[dst_v.at[last]], add=True)
        return carry

    lax.fori_loop(0, NBLK, blk, 0)
    plsc.subcore_barrier()
    pltpu.sync_copy(acc.at[pl.ds(s * RPT, RPT)], out_hbm.at[c, s])


BR = 400
GRID = N // BR


def _mm_body(x_ref, w_ref, o_ref):
    o_ref[...] = jnp.dot(x_ref[...], w_ref[...],
                         preferred_element_type=jnp.float32)


def _matmul(x, w):
    return pl.pallas_call(
        _mm_body,
        grid=(GRID,),
        in_specs=[
            pl.BlockSpec((BR, F), lambda i: (i, 0)),
            pl.BlockSpec((F, F), lambda i: (0, 0)),
        ],
        out_specs=pl.BlockSpec((BR, F), lambda i: (i, 0)),
        out_shape=jax.ShapeDtypeStruct((N, F), jnp.float32),
    )(x, w)


def _scale_body(p_ref, d0_ref, d1_ref, q_ref, dv_ref):
    deg = 1.0 + d0_ref[...] + d1_ref[...]
    dinv = lax.rsqrt(deg)
    dv_ref[...] = dinv[:, 0:DEG_W]
    q_ref[...] = p_ref[...] * dinv


def _scale(p, d0, d1):
    return pl.pallas_call(
        _scale_body,
        grid=(GRID,),
        in_specs=[
            pl.BlockSpec((BR, F), lambda i: (i, 0)),
            pl.BlockSpec((BR, F), lambda i: (i, 0)),
            pl.BlockSpec((BR, F), lambda i: (i, 0)),
        ],
        out_specs=[
            pl.BlockSpec((BR, F), lambda i: (i, 0)),
            pl.BlockSpec((BR, DEG_W), lambda i: (i, 0)),
        ],
        out_shape=[
            jax.ShapeDtypeStruct((N, F), jnp.float32),
            jax.ShapeDtypeStruct((N, DEG_W), jnp.float32),
        ],
    )(p, d0, d1)


def _mid_body(s0_ref, s1_ref, q_ref, dv_ref, b_ref, w_ref, o_ref):
    dv = dv_ref[:, 0:1]
    h = dv * (s0_ref[...] + s1_ref[...] + q_ref[...]) + b_ref[...]
    h = jnp.maximum(h, 0.0)
    o_ref[...] = jnp.dot(h, w_ref[...],
                         preferred_element_type=jnp.float32) * dv


def _mid(s0, s1, q, dv, b, w):
    return pl.pallas_call(
        _mid_body,
        grid=(GRID,),
        in_specs=[
            pl.BlockSpec((BR, F), lambda i: (i, 0)),
            pl.BlockSpec((BR, F), lambda i: (i, 0)),
            pl.BlockSpec((BR, F), lambda i: (i, 0)),
            pl.BlockSpec((BR, DEG_W), lambda i: (i, 0)),
            pl.BlockSpec((1, F), lambda i: (0, 0)),
            pl.BlockSpec((F, F), lambda i: (0, 0)),
        ],
        out_specs=pl.BlockSpec((BR, F), lambda i: (i, 0)),
        out_shape=jax.ShapeDtypeStruct((N, F), jnp.float32),
    )(s0, s1, q, dv, b, w)


def _head_body(s0_ref, s1_ref, q_ref, dv_ref, b_ref, wc_ref, bc_ref,
               logits_ref, emb_ref, soft_ref, hard_ref):
    dv = dv_ref[:, 0:1]
    emb = dv * (s0_ref[...] + s1_ref[...] + q_ref[...]) + b_ref[...]
    emb_ref[...] = emb
    logits = jnp.dot(emb, wc_ref[...],
                     preferred_element_type=jnp.float32) + bc_ref[...]
    logits_ref[...] = logits
    m = jnp.max(logits, axis=1, keepdims=True)
    ex = jnp.exp(logits - m)
    soft = ex / jnp.sum(ex, axis=1, keepdims=True)
    soft_ref[...] = soft
    smax = jnp.max(soft, axis=1, keepdims=True)
    ids = lax.broadcasted_iota(jnp.int32, (BR, OUT_F), 1)
    cand = jnp.where(soft >= smax, ids, OUT_F)
    hard_ref[...] = jnp.min(cand, axis=1, keepdims=True)


def _head(s0, s1, q, dv, b, wc, bc):
    return pl.pallas_call(
        _head_body,
        grid=(GRID,),
        in_specs=[
            pl.BlockSpec((BR, F), lambda i: (i, 0)),
            pl.BlockSpec((BR, F), lambda i: (i, 0)),
            pl.BlockSpec((BR, F), lambda i: (i, 0)),
            pl.BlockSpec((BR, DEG_W), lambda i: (i, 0)),
            pl.BlockSpec((1, F), lambda i: (0, 0)),
            pl.BlockSpec((F, OUT_F), lambda i: (0, 0)),
            pl.BlockSpec((1, OUT_F), lambda i: (0, 0)),
        ],
        out_specs=[
            pl.BlockSpec((BR, OUT_F), lambda i: (i, 0)),
            pl.BlockSpec((BR, F), lambda i: (i, 0)),
            pl.BlockSpec((BR, OUT_F), lambda i: (i, 0)),
            pl.BlockSpec((BR, 1), lambda i: (i, 0)),
        ],
        out_shape=[
            jax.ShapeDtypeStruct((N, OUT_F), jnp.float32),
            jax.ShapeDtypeStruct((N, F), jnp.float32),
            jax.ShapeDtypeStruct((N, OUT_F), jnp.float32),
            jax.ShapeDtypeStruct((N, 1), jnp.int32),
        ],
    )(s0, s1, q, dv, b, wc, bc)


def kernel(x, edge_index, W1, b1, W2, b2, Wc, bc):
    src4d = edge_index[0].astype(jnp.int32).reshape(NW, NBLK, NB, CH)
    dst4d = edge_index[1].astype(jnp.int32).reshape(NW, NBLK, NB, CH)
    dst3d = edge_index[1].astype(jnp.int32).reshape(NW, NCHUNK, CH)
    b1r = b1.reshape(1, F)
    b2r = b2.reshape(1, F)
    bcr = bc.reshape(1, OUT_F)

    degp = _deg_kernel(dst3d).reshape(NC, N, F)
    p1 = _matmul(x, W1)
    q1, dv = _scale(p1, degp[0], degp[1])
    s1 = _agg_kernel(q1, src4d, dst4d).reshape(NC, N, F)
    q2 = _mid(s1[0], s1[1], q1, dv, b1r, W2)
    s2 = _agg_kernel(q2, src4d, dst4d).reshape(NC, N, F)
    logits, emb, soft, hard = _head(s2[0], s2[1], q2, dv, b2r, Wc, bcr)
    return (logits, emb, soft, hard.reshape(N))
